# baseline (speedup 1.0000x reference)
import jax
import jax.numpy as jnp
from jax.experimental import pallas as pl


def _copy_body(x_ref, o_ref):
    o_ref[...] = x_ref[...]


def kernel(input, index, src):
    cols = jnp.arange(input.shape[1], dtype=index.dtype)[None, :]
    cols = jnp.broadcast_to(cols, index.shape)
    out = input.at[index, cols].add(src)
    M, D = out.shape
    B = 4000
    return pl.pallas_call(
        _copy_body,
        grid=(M // B,),
        in_specs=[pl.BlockSpec((B, D), lambda i: (i, 0))],
        out_specs=pl.BlockSpec((B, D), lambda i: (i, 0)),
        out_shape=jax.ShapeDtypeStruct((M, D), jnp.int32),
    )(out)


# speedup vs baseline: 2.6117x; 2.6117x over previous
import functools

import jax
import jax.numpy as jnp
from jax import lax
from jax.experimental import pallas as pl
from jax.experimental.pallas import tpu as pltpu
from jax.experimental.pallas import tpu_sc as plsc

_M = 500000
_D = 64
_NSRC = 131072
_NE = _NSRC * _D
_NTILE = 16
_NCHUNK = 20
_RCHUNK = _M // _NCHUNK
_CE = _RCHUNK * _D
_INIT_SLICE = _CE // _NTILE
_PIECE = 5000
_SCAN_B = 8192
_NBATCH = _NE // _NTILE // _SCAN_B
_NVREG = _SCAN_B // 16
_FLUSHN = 2048
_CAP = _FLUSHN + 16


def _body(inp_hbm, idx_hbm, src_hbm, out_hbm,
          acc, zeros_sp, stage_idx, stage_src, offbuf, valbuf, stage_io):
    cid = lax.axis_index("c")
    sid = lax.axis_index("s")
    iota = lax.iota(jnp.int32, 16)

    def init_const(v, _):
        base = v * 16
        offbuf[pl.ds(base, 16)] = iota + base
        valbuf[pl.ds(base, 16)] = jnp.zeros((16,), jnp.int32)
        return 0

    lax.fori_loop(0, _CAP // 16, init_const, 0, unroll=4)

    @pl.when(sid == 0)
    def _():
        pltpu.sync_copy(valbuf, zeros_sp)

    plsc.subcore_barrier()

    def chunk_body(gg, _):
        g = cid * (_NCHUNK // 2) + gg
        ebase = g * _CE
        r0 = g * _RCHUNK
        r1 = r0 + _RCHUNK

        tbase = ebase + sid * _INIT_SLICE

        def init_piece(i, _):
            o = i * _PIECE
            pltpu.sync_copy(inp_hbm.at[pl.ds(tbase + o, _PIECE)], stage_io)
            pltpu.sync_copy(stage_io,
                            acc.at[pl.ds(sid * _INIT_SLICE + o, _PIECE)])
            return 0

        lax.fori_loop(0, _INIT_SLICE // _PIECE, init_piece, 0)
        plsc.subcore_barrier()

        def flush():
            pltpu.sync_copy(valbuf, acc.at[offbuf], add=True)
            pltpu.sync_copy(zeros_sp, valbuf)

        def batch_body(b, ptr):
            sbase = sid * (_NE // _NTILE) + b * _SCAN_B
            pltpu.sync_copy(idx_hbm.at[pl.ds(sbase, _SCAN_B)], stage_idx)
            pltpu.sync_copy(src_hbm.at[pl.ds(sbase, _SCAN_B)], stage_src)

            def vreg_body(v, ptr):
                rows = stage_idx[pl.ds(v * 16, 16)]
                vals = stage_src[pl.ds(v * 16, 16)]
                col = ((v & 3) << 4) + iota
                local = (rows - r0) * _D + col
                mask = (rows >= r0) & (rows < r1)
                cs = jnp.cumsum(jnp.where(mask, 1, 0))
                pos = ptr + cs - 1
                plsc.store_scatter(offbuf, [pos], local, mask=mask)
                plsc.store_scatter(valbuf, [pos], vals, mask=mask)
                ptr = ptr + cs[15]
                full = ptr >= _FLUSHN
                pl.when(full)(flush)
                return jnp.where(full, 0, ptr)

            return lax.fori_loop(0, _NVREG, vreg_body, ptr)

        ptr = lax.fori_loop(0, _NBATCH, batch_body, jnp.int32(0))
        pl.when(ptr > 0)(flush)
        plsc.subcore_barrier()

        def wb_piece(i, _):
            o = i * _PIECE
            pltpu.sync_copy(acc.at[pl.ds(sid * _INIT_SLICE + o, _PIECE)],
                            stage_io)
            pltpu.sync_copy(stage_io, out_hbm.at[pl.ds(tbase + o, _PIECE)])
            return 0

        lax.fori_loop(0, _INIT_SLICE // _PIECE, wb_piece, 0)
        plsc.subcore_barrier()
        return 0

    lax.fori_loop(0, _NCHUNK // 2, chunk_body, 0)


def kernel(input, index, src):
    mesh = plsc.VectorSubcoreMesh(core_axis_name="c", subcore_axis_name="s")
    k = pl.kernel(
        _body,
        out_type=jax.ShapeDtypeStruct((_M * _D,), jnp.int32),
        mesh=mesh,
        compiler_params=pltpu.CompilerParams(needs_layout_passes=False),
        scratch_types=[
            pltpu.VMEM_SHARED((_CE,), jnp.int32),
            pltpu.VMEM_SHARED((_CAP,), jnp.int32),
            pltpu.VMEM((_SCAN_B,), jnp.int32),
            pltpu.VMEM((_SCAN_B,), jnp.int32),
            pltpu.VMEM((_CAP,), jnp.int32),
            pltpu.VMEM((_CAP,), jnp.int32),
            pltpu.VMEM((_PIECE,), jnp.int32),
        ],
    )
    out = k(input.reshape(-1), index.reshape(-1), src.reshape(-1))
    return out.reshape(_M, _D)


# speedup vs baseline: 4.3517x; 1.6662x over previous
import jax
import jax.numpy as jnp
from jax import lax
from jax.experimental import pallas as pl
from jax.experimental.pallas import tpu as pltpu
from jax.experimental.pallas import tpu_sc as plsc

_M = 500000
_D = 64
_NSRC = 131072
_NE = _NSRC * _D
_NTILE = 16
_NCHUNK = 20
_RCHUNK = _M // _NCHUNK
_CE = _RCHUNK * _D
_INIT_SLICE = _CE // _NTILE
_PIECE = 4000
_NPIECE = _INIT_SLICE // _PIECE
_SCAN_B = 4096
_EPT = _NE // _NTILE
_NBATCH = _EPT // _SCAN_B
_GRP = 8
_NGRP = _SCAN_B // 16 // _GRP
_REG = 272
_CAP = _GRP * _REG
_FTH = _REG - 16


def _body(inp_hbm, idx_hbm, src_hbm, out_hbm,
          acc, zeros_sp, stage_idx, stage_src, offbuf, valbuf, stage_io,
          sem_in, sem_io):
    cid = lax.axis_index("c")
    sid = lax.axis_index("s")
    iota = lax.iota(jnp.int32, 16)

    def init_const(v, _):
        base = v * 16
        offbuf[pl.ds(base, 16)] = iota + base
        valbuf[pl.ds(base, 16)] = jnp.zeros((16,), jnp.int32)
        return 0

    lax.fori_loop(0, _CAP // 16, init_const, 0, unroll=4)

    @pl.when(sid == 0)
    def _():
        pltpu.sync_copy(valbuf, zeros_sp)

    plsc.subcore_barrier()

    def in_copies(b):
        p = lax.rem(b, 2)
        sbase = sid * _EPT + b * _SCAN_B
        return (
            pltpu.make_async_copy(idx_hbm.at[pl.ds(sbase, _SCAN_B)],
                                  stage_idx.at[pl.ds(p * _SCAN_B, _SCAN_B)],
                                  sem_in.at[p]),
            pltpu.make_async_copy(src_hbm.at[pl.ds(sbase, _SCAN_B)],
                                  stage_src.at[pl.ds(p * _SCAN_B, _SCAN_B)],
                                  sem_in.at[p]),
        )

    def chunk_body(gg, _):
        g = cid * (_NCHUNK // 2) + gg
        ebase = g * _CE
        sbase = sid * _EPT

        tbase = ebase + sid * _INIT_SLICE

        def init_piece(i, _):
            o = i * _PIECE
            pltpu.sync_copy(inp_hbm.at[pl.ds(tbase + o, _PIECE)],
                            stage_io.at[pl.ds(0, _PIECE)])
            pltpu.sync_copy(stage_io.at[pl.ds(0, _PIECE)],
                            acc.at[pl.ds(sid * _INIT_SLICE + o, _PIECE)])
            return 0

        lax.fori_loop(0, _NPIECE, init_piece, 0)
        plsc.subcore_barrier()

        colmb = [iota + (jj * 16 - ebase) for jj in range(4)]

        def flush():
            pltpu.sync_copy(valbuf, acc.at[offbuf], add=True)
            pltpu.sync_copy(zeros_sp, valbuf)

        def batch_body(b, fills):
            for c in in_copies(b):
                c.wait()

            @pl.when(b + 1 < _NBATCH)
            def _():
                for c in in_copies(b + 1):
                    c.start()

            vbase = lax.rem(b, 2) * _SCAN_B

            def group_body(grp, fills):
                m = fills[0]
                for k in range(1, _GRP):
                    m = jnp.maximum(m, fills[k])
                full = m >= _FTH
                pl.when(full)(flush)
                fills = [jnp.where(full, 0, f) for f in fills]
                base = vbase + grp * (16 * _GRP)
                new_fills = []
                for j in range(_GRP):
                    rows = stage_idx[pl.ds(base + j * 16, 16)]
                    vals = stage_src[pl.ds(base + j * 16, 16)]
                    local = (rows << 6) + colmb[j % 4]
                    mask = local.astype(jnp.uint32) < jnp.uint32(_CE)
                    ones = jnp.where(mask, 1, 0)
                    cs = jnp.cumsum(ones)
                    pos = cs + (fills[j] + (j * _REG - 1))
                    plsc.store_scatter(offbuf, [pos], local, mask=mask)
                    plsc.store_scatter(valbuf, [pos], vals, mask=mask)
                    new_fills.append(fills[j] + jnp.sum(ones))
                return new_fills

            return lax.fori_loop(0, _NGRP, group_body, fills)

        zero = jnp.int32(0)
        for c in in_copies(0):
            c.start()
        lax.fori_loop(0, _NBATCH, batch_body, [zero] * _GRP)
        flush()
        plsc.subcore_barrier()

        def wb_piece(i, _):
            o = i * _PIECE
            pltpu.sync_copy(acc.at[pl.ds(sid * _INIT_SLICE + o, _PIECE)],
                            stage_io.at[pl.ds(0, _PIECE)])
            pltpu.sync_copy(stage_io.at[pl.ds(0, _PIECE)],
                            out_hbm.at[pl.ds(tbase + o, _PIECE)])
            return 0

        lax.fori_loop(0, _NPIECE, wb_piece, 0)
        plsc.subcore_barrier()
        return 0

    lax.fori_loop(0, _NCHUNK // 2, chunk_body, 0)


def kernel(input, index, src):
    mesh = plsc.VectorSubcoreMesh(core_axis_name="c", subcore_axis_name="s")
    k = pl.kernel(
        _body,
        out_type=jax.ShapeDtypeStruct((_M * _D,), jnp.int32),
        mesh=mesh,
        compiler_params=pltpu.CompilerParams(needs_layout_passes=False),
        scratch_types=[
            pltpu.VMEM_SHARED((_CE,), jnp.int32),
            pltpu.VMEM_SHARED((_CAP,), jnp.int32),
            pltpu.VMEM((2 * _SCAN_B,), jnp.int32),
            pltpu.VMEM((2 * _SCAN_B,), jnp.int32),
            pltpu.VMEM((_CAP,), jnp.int32),
            pltpu.VMEM((_CAP,), jnp.int32),
            pltpu.VMEM((_PIECE,), jnp.int32),
            pltpu.SemaphoreType.DMA((2,)),
            pltpu.SemaphoreType.DMA((2,)),
        ],
    )
    out = k(input.reshape(-1), index.reshape(-1), src.reshape(-1))
    return out.reshape(_M, _D)


# speedup vs baseline: 9.5387x; 2.1920x over previous
import jax
import jax.numpy as jnp
from jax import lax
from jax.experimental import pallas as pl
from jax.experimental.pallas import tpu as pltpu
from jax.experimental.pallas import tpu_sc as plsc

_M = 500000
_D = 64
_NSRC = 131072
_NE = _NSRC * _D
_NTILE = 16
_NCHUNK = 20
_RCHUNK = _M // _NCHUNK
_CE = _RCHUNK * _D
_INIT_SLICE = _CE // _NTILE
_PIECE = 4000
_NPIECE = _INIT_SLICE // _PIECE
_SCAN_B = 4096
_EPT = _NE // _NTILE
_NBATCH = _EPT // _SCAN_B
_GRP = 8
_NGRP = _SCAN_B // 16 // _GRP
_REG = 272
_CAP = _GRP * _REG
_FTH = _REG - 16


def _body(inp_hbm, idx_hbm, src_hbm, out_hbm,
          acc, zeros_sp, stage_idx, stage_src, offbuf, valbuf, stage_io,
          sem_in, sem_io):
    cid = lax.axis_index("c")
    sid = lax.axis_index("s")
    iota = lax.iota(jnp.int32, 16)

    def init_const(v, _):
        base = v * 16
        offbuf[pl.ds(base, 16)] = iota + base
        valbuf[pl.ds(base, 16)] = jnp.zeros((16,), jnp.int32)
        return 0

    lax.fori_loop(0, _CAP // 16, init_const, 0, unroll=4)

    @pl.when(sid == 0)
    def _():
        pltpu.sync_copy(valbuf, zeros_sp)

    plsc.subcore_barrier()

    def in_copies(b):
        p = lax.rem(b, 2)
        sbase = sid * _EPT + b * _SCAN_B
        return (
            pltpu.make_async_copy(idx_hbm.at[pl.ds(sbase, _SCAN_B)],
                                  stage_idx.at[pl.ds(p * _SCAN_B, _SCAN_B)],
                                  sem_in.at[p]),
            pltpu.make_async_copy(src_hbm.at[pl.ds(sbase, _SCAN_B)],
                                  stage_src.at[pl.ds(p * _SCAN_B, _SCAN_B)],
                                  sem_in.at[p]),
        )

    def chunk_body(gg, _):
        g = cid * (_NCHUNK // 2) + gg
        ebase = g * _CE
        sbase = sid * _EPT

        tbase = ebase + sid * _INIT_SLICE

        def init_piece(i, _):
            o = i * _PIECE
            pltpu.sync_copy(inp_hbm.at[pl.ds(tbase + o, _PIECE)],
                            stage_io.at[pl.ds(0, _PIECE)])
            pltpu.sync_copy(stage_io.at[pl.ds(0, _PIECE)],
                            acc.at[pl.ds(sid * _INIT_SLICE + o, _PIECE)])
            return 0

        lax.fori_loop(0, _NPIECE, init_piece, 0)
        plsc.subcore_barrier()

        colmb = [iota + (jj * 16 - ebase) for jj in range(4)]

        def flush():
            pltpu.sync_copy(valbuf, acc.at[offbuf], add=True)
            pltpu.sync_copy(zeros_sp, valbuf)

        def batch_body(b, fills):
            for c in in_copies(b):
                c.wait()

            @pl.when(b + 1 < _NBATCH)
            def _():
                for c in in_copies(b + 1):
                    c.start()

            vbase = lax.rem(b, 2) * _SCAN_B

            f15 = jnp.full((16,), 15, jnp.int32)
            neg1 = jnp.full((16,), -1, jnp.int32)

            def group_body(grp, fills):
                m = fills[0]
                for k in range(1, _GRP):
                    m = jnp.maximum(m, fills[k])
                full = jnp.max(m) >= _FTH
                pl.when(full)(flush)
                fills = [jnp.where(full, neg1, f) for f in fills]
                base = vbase + grp * (16 * _GRP)
                rows_l = [stage_idx[pl.ds(base + j * 16, 16)]
                          for j in range(_GRP)]
                vals_l = [stage_src[pl.ds(base + j * 16, 16)]
                          for j in range(_GRP)]
                local_l = [(rows_l[j] << 6) + colmb[j % 4]
                           for j in range(_GRP)]
                mask_l = [local_l[j].astype(jnp.uint32) < jnp.uint32(_CE)
                          for j in range(_GRP)]
                cs_l = [jnp.cumsum(jnp.where(mask_l[j], 1, 0))
                        for j in range(_GRP)]
                pos0_l = [cs_l[j] + fills[j] for j in range(_GRP)]
                new_fills = [pos0_l[j].at[f15].get(mode="promise_in_bounds")
                             for j in range(_GRP)]
                for j in range(_GRP):
                    pos = pos0_l[j] + (j * _REG)
                    plsc.store_scatter(offbuf, [pos], local_l[j],
                                       mask=mask_l[j])
                    plsc.store_scatter(valbuf, [pos], vals_l[j],
                                       mask=mask_l[j])
                return new_fills

            return lax.fori_loop(0, _NGRP, group_body, fills)

        for c in in_copies(0):
            c.start()
        lax.fori_loop(0, _NBATCH, batch_body,
                      [jnp.full((16,), -1, jnp.int32)] * _GRP)
        flush()
        plsc.subcore_barrier()

        def wb_piece(i, _):
            o = i * _PIECE
            pltpu.sync_copy(acc.at[pl.ds(sid * _INIT_SLICE + o, _PIECE)],
                            stage_io.at[pl.ds(0, _PIECE)])
            pltpu.sync_copy(stage_io.at[pl.ds(0, _PIECE)],
                            out_hbm.at[pl.ds(tbase + o, _PIECE)])
            return 0

        lax.fori_loop(0, _NPIECE, wb_piece, 0)
        plsc.subcore_barrier()
        return 0

    lax.fori_loop(0, _NCHUNK // 2, chunk_body, 0)


def kernel(input, index, src):
    mesh = plsc.VectorSubcoreMesh(core_axis_name="c", subcore_axis_name="s")
    k = pl.kernel(
        _body,
        out_type=jax.ShapeDtypeStruct((_M * _D,), jnp.int32),
        mesh=mesh,
        compiler_params=pltpu.CompilerParams(needs_layout_passes=False),
        scratch_types=[
            pltpu.VMEM_SHARED((_CE,), jnp.int32),
            pltpu.VMEM_SHARED((_CAP,), jnp.int32),
            pltpu.VMEM((2 * _SCAN_B,), jnp.int32),
            pltpu.VMEM((2 * _SCAN_B,), jnp.int32),
            pltpu.VMEM((_CAP,), jnp.int32),
            pltpu.VMEM((_CAP,), jnp.int32),
            pltpu.VMEM((_PIECE,), jnp.int32),
            pltpu.SemaphoreType.DMA((2,)),
            pltpu.SemaphoreType.DMA((2,)),
        ],
    )
    out = k(input.reshape(-1), index.reshape(-1), src.reshape(-1))
    return out.reshape(_M, _D)
